# baseline (speedup 1.0000x reference)
import functools

import jax
import jax.numpy as jnp
from jax import lax
from jax.experimental import pallas as pl
from jax.experimental.pallas import tpu as pltpu
from jax.experimental.pallas import tpu_sc as plsc

N_NODES = 10000
N_EDGES = 320000
D_FEAT = 128
NUM_TYPES = 4
NUM_BOND_TYPES = 16

NC = 2
NS = 16
NW = NC * NS
LANES = 16

CHUNK = 80
N_NODE_CHUNKS = N_NODES // CHUNK
N_EDGE_CHUNKS = N_EDGES // CHUNK
EDGE_CHUNKS_PER_W = N_EDGE_CHUNKS // NW
NODE_K = (N_NODE_CHUNKS + NW - 1) // NW


NCH = D_FEAT // LANES


def _zero_acc(accs, rows):
    z = jnp.zeros((LANES,), jnp.float32)
    for acc in accs:
        for t in range(rows):
            acc[t] = z


def _row_loop(f_buf, r_buf, t_buf, acc_abs, acc_sq, cnt):
    lane_iota = lax.iota(jnp.int32, LANES)

    def body(g, carry):
        tv = t_buf[pl.ds(g * LANES, LANES)]
        for j in range(LANES):
            t = tv[j]
            r = g * LANES + j
            for c in range(NCH):
                sl = pl.ds(c * LANES, LANES)
                d = f_buf[r, sl] - r_buf[r, sl]
                plsc.addupdate(acc_abs[c].at[t], jnp.abs(d))
                plsc.addupdate(acc_sq[c].at[t], d * d)
            plsc.addupdate(cnt.at[...], jnp.where(lane_iota == t, 1.0, 0.0))
        return carry

    lax.fori_loop(0, CHUNK // LANES, body, 0)


def _sc_partials(nf, rnf, ef, ref_, at, et):
    mesh = plsc.VectorSubcoreMesh(core_axis_name="c", subcore_axis_name="s")

    @functools.partial(
        pl.kernel,
        out_type=(
            jax.ShapeDtypeStruct((NW, NUM_TYPES, D_FEAT), jnp.float32),
            jax.ShapeDtypeStruct((NW, NUM_TYPES, D_FEAT), jnp.float32),
            jax.ShapeDtypeStruct((NW, NUM_BOND_TYPES, D_FEAT), jnp.float32),
            jax.ShapeDtypeStruct((NW, NUM_BOND_TYPES, D_FEAT), jnp.float32),
            jax.ShapeDtypeStruct((NW, 2 * LANES), jnp.float32),
        ),
        mesh=mesh,
        scratch_types=(
            [pltpu.VMEM((CHUNK, D_FEAT), jnp.float32),
             pltpu.VMEM((CHUNK, D_FEAT), jnp.float32),
             pltpu.VMEM((CHUNK,), jnp.int32)]
            + [pltpu.VMEM((NUM_TYPES, LANES), jnp.float32)] * (2 * NCH)
            + [pltpu.VMEM((NUM_BOND_TYPES, LANES), jnp.float32)] * (2 * NCH)
            + [pltpu.VMEM((LANES,), jnp.float32)] * 2
        ),
    )
    def k(nf_h, rnf_h, ef_h, ref_h, at_h, et_h,
          o_nabs, o_nsq, o_eabs, o_esq, o_cnt,
          f_buf, r_buf, t_buf, *accs):
        a_nabs = list(accs[0:NCH])
        a_nsq = list(accs[NCH:2 * NCH])
        a_eabs = list(accs[2 * NCH:3 * NCH])
        a_esq = list(accs[3 * NCH:4 * NCH])
        c_n, c_e = accs[4 * NCH], accs[4 * NCH + 1]
        w = lax.axis_index("s") * NC + lax.axis_index("c")

        _zero_acc(a_nabs, NUM_TYPES)
        _zero_acc(a_nsq, NUM_TYPES)
        _zero_acc(a_eabs, NUM_BOND_TYPES)
        _zero_acc(a_esq, NUM_BOND_TYPES)
        z = jnp.zeros((LANES,), jnp.float32)
        c_n[...] = z
        c_e[...] = z

        for kk in range(NODE_K):
            chunk = kk * NW + w

            @pl.when(chunk < N_NODE_CHUNKS)
            def _():
                start = pl.multiple_of(chunk * CHUNK, 8)
                pltpu.sync_copy(nf_h.at[pl.ds(start, CHUNK)], f_buf)
                pltpu.sync_copy(rnf_h.at[pl.ds(start, CHUNK)], r_buf)
                pltpu.sync_copy(at_h.at[pl.ds(start, CHUNK)], t_buf)
                _row_loop(f_buf, r_buf, t_buf, a_nabs, a_nsq, c_n)

        def edge_body(kk, carry):
            chunk = w * EDGE_CHUNKS_PER_W + kk
            start = pl.multiple_of(chunk * CHUNK, 8)
            pltpu.sync_copy(ef_h.at[pl.ds(start, CHUNK)], f_buf)
            pltpu.sync_copy(ref_h.at[pl.ds(start, CHUNK)], r_buf)
            pltpu.sync_copy(et_h.at[pl.ds(start, CHUNK)], t_buf)
            _row_loop(f_buf, r_buf, t_buf, a_eabs, a_esq, c_e)
            return carry

        lax.fori_loop(0, EDGE_CHUNKS_PER_W, edge_body, 0)

        row = 0
        for accs_group, rows in ((a_nabs, NUM_TYPES), (a_nsq, NUM_TYPES),
                                 (a_eabs, NUM_BOND_TYPES),
                                 (a_esq, NUM_BOND_TYPES)):
            for t in range(rows):
                for c in range(NCH):
                    f_buf[row + t, pl.ds(c * LANES, LANES)] = accs_group[c][t]
            row += rows
        pltpu.sync_copy(f_buf.at[pl.ds(0, NUM_TYPES)], o_nabs.at[w])
        pltpu.sync_copy(f_buf.at[pl.ds(NUM_TYPES, NUM_TYPES)], o_nsq.at[w])
        pltpu.sync_copy(f_buf.at[pl.ds(8, NUM_BOND_TYPES)], o_eabs.at[w])
        pltpu.sync_copy(f_buf.at[pl.ds(24, NUM_BOND_TYPES)], o_esq.at[w])
        pltpu.sync_copy(c_n, o_cnt.at[w, pl.ds(0, LANES)])
        pltpu.sync_copy(c_e, o_cnt.at[w, pl.ds(LANES, LANES)])

    return k(nf, rnf, ef, ref_, at, et)


def _tc_combine_body(pn_abs, pn_sq, pe_abs, pe_sq, pcnt, nm, em, ow, hw, out):
    nabs = jnp.sum(pn_abs[...], axis=0)
    nsq = jnp.sum(pn_sq[...], axis=0)
    eabs = jnp.sum(pe_abs[...], axis=0)
    esq = jnp.sum(pe_sq[...], axis=0)
    cnt = jnp.sum(pcnt[...], axis=0)
    cnt_n = cnt[0:NUM_TYPES]
    cnt_e = cnt[LANES:LANES + NUM_BOND_TYPES]

    nmf = nm[...]
    emf = em[...]
    wn = ow[...][:, 0]
    we = hw[...][:, 0]

    inv_cn = wn / cnt_n
    inv_cn2 = wn * wn / cnt_n
    inv_ce = we / cnt_e
    inv_ce2 = we * we / cnt_e

    s_abs_n = jnp.sum(nabs * nmf, axis=1)
    s_sq_n = jnp.sum(nsq * nmf, axis=1)
    s_abs_e = jnp.sum(eabs * emf, axis=1)
    s_sq_e = jnp.sum(esq * emf, axis=1)

    msum_n = jnp.sum(nmf)
    msum_e = jnp.sum(emf)

    mm_abs_n = jnp.sum(s_abs_n * inv_cn) / msum_n
    mm_sq_n = jnp.sum(s_sq_n * inv_cn2) / msum_n
    mm_abs_e = jnp.sum(s_abs_e * inv_ce) / msum_e
    mm_sq_e = jnp.sum(s_sq_e * inv_ce2) / msum_e

    onsite = mm_abs_n + jnp.sqrt(mm_sq_n)
    hopping = mm_abs_e + jnp.sqrt(mm_sq_e)
    total = 0.25 * (onsite + hopping)
    out[...] = jnp.full((1, 1), total, jnp.float32)


def kernel(node_features, ref_node_features, edge_features, ref_edge_features,
           atom_type, edge_type, onsite_weight, hopping_weight,
           mask_to_nrme, mask_to_erme):
    pn_abs, pn_sq, pe_abs, pe_sq, pcnt = _sc_partials(
        node_features, ref_node_features, edge_features, ref_edge_features,
        atom_type, edge_type)

    out = pl.pallas_call(
        _tc_combine_body,
        out_shape=jax.ShapeDtypeStruct((1, 1), jnp.float32),
    )(pn_abs, pn_sq, pe_abs, pe_sq, pcnt,
      mask_to_nrme.astype(jnp.float32), mask_to_erme.astype(jnp.float32),
      onsite_weight, hopping_weight)
    return out[0, 0]


# speedup vs baseline: 1.6063x; 1.6063x over previous
import functools

import jax
import jax.numpy as jnp
from jax import lax
from jax.experimental import pallas as pl
from jax.experimental.pallas import tpu as pltpu
from jax.experimental.pallas import tpu_sc as plsc

N_NODES = 10000
N_EDGES = 320000
D_FEAT = 128
NUM_TYPES = 4
NUM_BOND_TYPES = 16

NC = 2
NS = 16
NW = NC * NS
LANES = 16

CHUNK = 80
N_NODE_CHUNKS = N_NODES // CHUNK
N_EDGE_CHUNKS = N_EDGES // CHUNK
EDGE_CHUNKS_PER_W = N_EDGE_CHUNKS // NW
NODE_K = (N_NODE_CHUNKS + NW - 1) // NW


NCH = D_FEAT // LANES


def _zero_acc(accs, rows):
    z = jnp.zeros((LANES,), jnp.float32)
    for acc in accs:
        for t in range(rows):
            acc[t] = z


def _row_loop(f_buf, r_buf, t_buf, acc_abs, acc_sq, cnt):
    lane_iota = lax.iota(jnp.int32, LANES)

    def body(g, carry):
        tv = t_buf[pl.ds(g * LANES, LANES)]
        for j in range(LANES):
            t = tv[j]
            r = g * LANES + j
            abss, sqs = [], []
            for c in range(NCH):
                sl = pl.ds(c * LANES, LANES)
                d = f_buf[r, sl] - r_buf[r, sl]
                abss.append(jnp.abs(d))
                sqs.append(d * d)
            for c in range(NCH):
                plsc.addupdate(acc_abs[c].at[t], abss[c])
                plsc.addupdate(acc_sq[c].at[t], sqs[c])
            plsc.addupdate(cnt.at[...], jnp.where(lane_iota == t, 1.0, 0.0))
        return carry

    lax.fori_loop(0, CHUNK // LANES, body, 0)


def _sc_partials(nf, rnf, ef, ref_, at, et):
    mesh = plsc.VectorSubcoreMesh(core_axis_name="c", subcore_axis_name="s")

    @functools.partial(
        pl.kernel,
        out_type=(
            jax.ShapeDtypeStruct((NW, NUM_TYPES, D_FEAT), jnp.float32),
            jax.ShapeDtypeStruct((NW, NUM_TYPES, D_FEAT), jnp.float32),
            jax.ShapeDtypeStruct((NW, NUM_BOND_TYPES, D_FEAT), jnp.float32),
            jax.ShapeDtypeStruct((NW, NUM_BOND_TYPES, D_FEAT), jnp.float32),
            jax.ShapeDtypeStruct((NW, 2 * LANES), jnp.float32),
        ),
        mesh=mesh,
        scratch_types=(
            [pltpu.VMEM((CHUNK, D_FEAT), jnp.float32),
             pltpu.VMEM((CHUNK, D_FEAT), jnp.float32),
             pltpu.VMEM((CHUNK,), jnp.int32)]
            + [pltpu.VMEM((NUM_TYPES, LANES), jnp.float32)] * (2 * NCH)
            + [pltpu.VMEM((NUM_BOND_TYPES, LANES), jnp.float32)] * (2 * NCH)
            + [pltpu.VMEM((LANES,), jnp.float32)] * 2
        ),
    )
    def k(nf_h, rnf_h, ef_h, ref_h, at_h, et_h,
          o_nabs, o_nsq, o_eabs, o_esq, o_cnt,
          f_buf, r_buf, t_buf, *accs):
        a_nabs = list(accs[0:NCH])
        a_nsq = list(accs[NCH:2 * NCH])
        a_eabs = list(accs[2 * NCH:3 * NCH])
        a_esq = list(accs[3 * NCH:4 * NCH])
        c_n, c_e = accs[4 * NCH], accs[4 * NCH + 1]
        w = lax.axis_index("s") * NC + lax.axis_index("c")

        _zero_acc(a_nabs, NUM_TYPES)
        _zero_acc(a_nsq, NUM_TYPES)
        _zero_acc(a_eabs, NUM_BOND_TYPES)
        _zero_acc(a_esq, NUM_BOND_TYPES)
        z = jnp.zeros((LANES,), jnp.float32)
        c_n[...] = z
        c_e[...] = z

        for kk in range(NODE_K):
            chunk = kk * NW + w

            @pl.when(chunk < N_NODE_CHUNKS)
            def _():
                start = pl.multiple_of(chunk * CHUNK, 8)
                pltpu.sync_copy(nf_h.at[pl.ds(start, CHUNK)], f_buf)
                pltpu.sync_copy(rnf_h.at[pl.ds(start, CHUNK)], r_buf)
                pltpu.sync_copy(at_h.at[pl.ds(start, CHUNK)], t_buf)
                _row_loop(f_buf, r_buf, t_buf, a_nabs, a_nsq, c_n)

        def edge_body(kk, carry):
            chunk = w * EDGE_CHUNKS_PER_W + kk
            start = pl.multiple_of(chunk * CHUNK, 8)
            pltpu.sync_copy(ef_h.at[pl.ds(start, CHUNK)], f_buf)
            pltpu.sync_copy(ref_h.at[pl.ds(start, CHUNK)], r_buf)
            pltpu.sync_copy(et_h.at[pl.ds(start, CHUNK)], t_buf)
            _row_loop(f_buf, r_buf, t_buf, a_eabs, a_esq, c_e)
            return carry

        lax.fori_loop(0, EDGE_CHUNKS_PER_W, edge_body, 0)

        row = 0
        for accs_group, rows in ((a_nabs, NUM_TYPES), (a_nsq, NUM_TYPES),
                                 (a_eabs, NUM_BOND_TYPES),
                                 (a_esq, NUM_BOND_TYPES)):
            for t in range(rows):
                for c in range(NCH):
                    f_buf[row + t, pl.ds(c * LANES, LANES)] = accs_group[c][t]
            row += rows
        pltpu.sync_copy(f_buf.at[pl.ds(0, NUM_TYPES)], o_nabs.at[w])
        pltpu.sync_copy(f_buf.at[pl.ds(NUM_TYPES, NUM_TYPES)], o_nsq.at[w])
        pltpu.sync_copy(f_buf.at[pl.ds(8, NUM_BOND_TYPES)], o_eabs.at[w])
        pltpu.sync_copy(f_buf.at[pl.ds(24, NUM_BOND_TYPES)], o_esq.at[w])
        pltpu.sync_copy(c_n, o_cnt.at[w, pl.ds(0, LANES)])
        pltpu.sync_copy(c_e, o_cnt.at[w, pl.ds(LANES, LANES)])

    return k(nf, rnf, ef, ref_, at, et)


def _tc_combine_body(pn_abs, pn_sq, pe_abs, pe_sq, pcnt, nm, em, ow, hw, out):
    nabs = jnp.sum(pn_abs[...], axis=0)
    nsq = jnp.sum(pn_sq[...], axis=0)
    eabs = jnp.sum(pe_abs[...], axis=0)
    esq = jnp.sum(pe_sq[...], axis=0)
    cnt = jnp.sum(pcnt[...], axis=0)
    cnt_n = cnt[0:NUM_TYPES]
    cnt_e = cnt[LANES:LANES + NUM_BOND_TYPES]

    nmf = nm[...]
    emf = em[...]
    wn = ow[...][:, 0]
    we = hw[...][:, 0]

    inv_cn = wn / cnt_n
    inv_cn2 = wn * wn / cnt_n
    inv_ce = we / cnt_e
    inv_ce2 = we * we / cnt_e

    s_abs_n = jnp.sum(nabs * nmf, axis=1)
    s_sq_n = jnp.sum(nsq * nmf, axis=1)
    s_abs_e = jnp.sum(eabs * emf, axis=1)
    s_sq_e = jnp.sum(esq * emf, axis=1)

    msum_n = jnp.sum(nmf)
    msum_e = jnp.sum(emf)

    mm_abs_n = jnp.sum(s_abs_n * inv_cn) / msum_n
    mm_sq_n = jnp.sum(s_sq_n * inv_cn2) / msum_n
    mm_abs_e = jnp.sum(s_abs_e * inv_ce) / msum_e
    mm_sq_e = jnp.sum(s_sq_e * inv_ce2) / msum_e

    onsite = mm_abs_n + jnp.sqrt(mm_sq_n)
    hopping = mm_abs_e + jnp.sqrt(mm_sq_e)
    total = 0.25 * (onsite + hopping)
    out[...] = jnp.full((1, 1), total, jnp.float32)


def kernel(node_features, ref_node_features, edge_features, ref_edge_features,
           atom_type, edge_type, onsite_weight, hopping_weight,
           mask_to_nrme, mask_to_erme):
    pn_abs, pn_sq, pe_abs, pe_sq, pcnt = _sc_partials(
        node_features, ref_node_features, edge_features, ref_edge_features,
        atom_type, edge_type)

    out = pl.pallas_call(
        _tc_combine_body,
        out_shape=jax.ShapeDtypeStruct((1, 1), jnp.float32),
    )(pn_abs, pn_sq, pe_abs, pe_sq, pcnt,
      mask_to_nrme.astype(jnp.float32), mask_to_erme.astype(jnp.float32),
      onsite_weight, hopping_weight)
    return out[0, 0]
